# baseline (speedup 1.0000x reference)
import functools

import jax
import jax.numpy as jnp
from jax import lax
from jax.experimental import pallas as pl
from jax.experimental.pallas import tpu as pltpu
from jax.experimental.pallas import tpu_sc as plsc

NUM_SERIES_C = 128
C = 128
NBUF = 4

SC_NC = 2
SC_NS = 16
SC_L = 16
PER_W = 128


def _sc_gather_body(i_hbm, loc_hbm, scale_hbm, lgrg_hbm,
                    idx_v, loc_v, scale_v, lg_v, rg_v):
    nw = SC_NC * SC_NS
    wid = lax.axis_index("s") * SC_NC + lax.axis_index("c")
    pltpu.sync_copy(i_hbm.at[wid], idx_v)
    pltpu.sync_copy(loc_hbm, loc_v)
    pltpu.sync_copy(scale_hbm, scale_v)
    for j in range(PER_W // SC_L):
        iv = idx_v[pl.ds(j * SC_L, SC_L)]
        lg_v[pl.ds(j * SC_L, SC_L)] = plsc.load_gather(loc_v, [iv])
        rg_v[pl.ds(j * SC_L, SC_L)] = 1.0 / plsc.load_gather(scale_v, [iv])
    pltpu.sync_copy(lg_v, lgrg_hbm.at[wid])
    pltpu.sync_copy(rg_v, lgrg_hbm.at[nw + wid])


def _tc_stream_body(lgrg_ref, x_hbm, o_hbm,
                    in_buf, out_buf, sem_in, sem_out):
    num_chunks = x_hbm.shape[0] // C

    def in_copy(c, j):
        return pltpu.make_async_copy(
            x_hbm.at[pl.ds(c * C, C), :, :],
            in_buf.at[pl.ds(j * C, C), :, :],
            sem_in.at[j],
        )

    def out_copy(c, j):
        return pltpu.make_async_copy(
            out_buf.at[pl.ds(j * C, C), :, :],
            o_hbm.at[pl.ds(c * C, C), :, :],
            sem_out.at[j],
        )

    for j in range(NBUF):
        in_copy(j, j).start()

    def step(c, _):
        j = lax.rem(c, NBUF)
        in_copy(c, j).wait()

        lgr = lgrg_ref[c, :]
        rgr = lgrg_ref[num_chunks + c, :]

        @pl.when(c >= NBUF)
        def _():
            out_copy(c - NBUF, j).wait()

        xin = in_buf[pl.ds(j * C, C), :, :]
        out_buf[pl.ds(j * C, C), :, :] = (
            xin - lgr[:, None, None]
        ) * rgr[:, None, None]
        out_copy(c, j).start()

        @pl.when(c + NBUF < num_chunks)
        def _():
            in_copy(c + NBUF, j).start()

        return _

    lax.fori_loop(0, num_chunks, step, None)

    for j in range(NBUF):
        out_copy(num_chunks - NBUF + j, j).wait()


def kernel(x, i, loc, scale):
    bs, num_patch, out_len = x.shape
    num_chunks = bs // C
    num_workers = SC_NC * SC_NS
    i2 = i.reshape(num_workers, PER_W)

    sc_gather = functools.partial(
        pl.kernel,
        out_type=jax.ShapeDtypeStruct((2 * num_workers, PER_W), jnp.float32),
        mesh=plsc.VectorSubcoreMesh(
            core_axis_name="c", subcore_axis_name="s"
        ),
        compiler_params=pltpu.CompilerParams(needs_layout_passes=False),
        scratch_types=[
            pltpu.VMEM((PER_W,), jnp.int32),
            pltpu.VMEM((NUM_SERIES_C,), jnp.float32),
            pltpu.VMEM((NUM_SERIES_C,), jnp.float32),
            pltpu.VMEM((PER_W,), jnp.float32),
            pltpu.VMEM((PER_W,), jnp.float32),
        ],
    )(_sc_gather_body)
    lgrg = sc_gather(i2, loc.reshape(-1), scale.reshape(-1))

    return pl.pallas_call(
        _tc_stream_body,
        in_specs=[
            pl.BlockSpec(memory_space=pltpu.MemorySpace.VMEM),
            pl.BlockSpec(memory_space=pltpu.MemorySpace.HBM),
        ],
        out_specs=pl.BlockSpec(memory_space=pltpu.MemorySpace.HBM),
        out_shape=jax.ShapeDtypeStruct((bs, num_patch, out_len), x.dtype),
        scratch_shapes=[
            pltpu.VMEM((NBUF * C, num_patch, out_len), x.dtype),
            pltpu.VMEM((NBUF * C, num_patch, out_len), x.dtype),
            pltpu.SemaphoreType.DMA((NBUF,)),
            pltpu.SemaphoreType.DMA((NBUF,)),
        ],
    )(lgrg, x)


# speedup vs baseline: 1.0011x; 1.0011x over previous
import functools

import jax
import jax.numpy as jnp
from jax import lax
from jax.experimental import pallas as pl
from jax.experimental.pallas import tpu as pltpu
from jax.experimental.pallas import tpu_sc as plsc

NUM_SERIES_C = 128
C = 128
NBUF = 4

SC_NC = 2
SC_NS = 16
SC_L = 16
PER_W = 128


def _sc_gather_body(i_hbm, loc_hbm, scale_hbm, lgrg_hbm,
                    idx_v, loc_v, scale_v, lg_v, rg_v):
    nw = SC_NC * SC_NS
    wid = lax.axis_index("s") * SC_NC + lax.axis_index("c")
    pltpu.sync_copy(i_hbm.at[wid], idx_v)
    pltpu.sync_copy(loc_hbm, loc_v)
    pltpu.sync_copy(scale_hbm, scale_v)
    for j in range(PER_W // SC_L):
        iv = idx_v[pl.ds(j * SC_L, SC_L)]
        lg_v[pl.ds(j * SC_L, SC_L)] = plsc.load_gather(loc_v, [iv])
        rg_v[pl.ds(j * SC_L, SC_L)] = 1.0 / plsc.load_gather(scale_v, [iv])
    pltpu.sync_copy(lg_v, lgrg_hbm.at[wid])
    pltpu.sync_copy(rg_v, lgrg_hbm.at[nw + wid])


def _tc_stream_body(lgrg_ref, x_hbm, o_hbm,
                    in_buf, out_buf, sem_in, sem_out):
    num_chunks = x_hbm.shape[0] // C

    def in_copy(c, j):
        return pltpu.make_async_copy(
            x_hbm.at[pl.ds(c * C, C), :, :],
            in_buf.at[pl.ds(j * C, C), :, :],
            sem_in.at[j],
        )

    def out_copy(c, j):
        return pltpu.make_async_copy(
            out_buf.at[pl.ds(j * C, C), :, :],
            o_hbm.at[pl.ds(c * C, C), :, :],
            sem_out.at[j],
        )

    for j in range(NBUF):
        in_copy(j, j).start()

    def step(c, _):
        j = lax.rem(c, NBUF)
        in_copy(c, j).wait()

        lgr = lgrg_ref[c, :]
        rgr = lgrg_ref[num_chunks + c, :]

        @pl.when(c >= NBUF)
        def _():
            out_copy(c - NBUF, j).wait()

        xin = in_buf[pl.ds(j * C, C), :, :]
        out_buf[pl.ds(j * C, C), :, :] = (
            xin - lgr[:, None, None]
        ) * rgr[:, None, None]
        out_copy(c, j).start()

        @pl.when(c + NBUF < num_chunks)
        def _():
            in_copy(c + NBUF, j).start()

        return _

    lax.fori_loop(0, num_chunks, step, None)

    for j in range(NBUF):
        out_copy(num_chunks - NBUF + j, j).wait()


def kernel(x, i, loc, scale):
    bs, num_patch, out_len = x.shape
    num_chunks = bs // C
    num_workers = SC_NC * SC_NS
    i2 = i.reshape(num_workers, PER_W)

    sc_gather = functools.partial(
        pl.kernel,
        out_type=jax.ShapeDtypeStruct((2 * num_workers, PER_W), jnp.float32),
        mesh=plsc.VectorSubcoreMesh(
            core_axis_name="c", subcore_axis_name="s"
        ),
        compiler_params=pltpu.CompilerParams(
            needs_layout_passes=False, skip_device_barrier=True
        ),
        scratch_types=[
            pltpu.VMEM((PER_W,), jnp.int32),
            pltpu.VMEM((NUM_SERIES_C,), jnp.float32),
            pltpu.VMEM((NUM_SERIES_C,), jnp.float32),
            pltpu.VMEM((PER_W,), jnp.float32),
            pltpu.VMEM((PER_W,), jnp.float32),
        ],
    )(_sc_gather_body)
    lgrg = sc_gather(i2, loc.reshape(-1), scale.reshape(-1))

    return pl.pallas_call(
        _tc_stream_body,
        in_specs=[
            pl.BlockSpec(memory_space=pltpu.MemorySpace.VMEM),
            pl.BlockSpec(memory_space=pltpu.MemorySpace.HBM),
        ],
        out_specs=pl.BlockSpec(memory_space=pltpu.MemorySpace.HBM),
        out_shape=jax.ShapeDtypeStruct((bs, num_patch, out_len), x.dtype),
        compiler_params=pltpu.CompilerParams(skip_device_barrier=True),
        scratch_shapes=[
            pltpu.VMEM((NBUF * C, num_patch, out_len), x.dtype),
            pltpu.VMEM((NBUF * C, num_patch, out_len), x.dtype),
            pltpu.SemaphoreType.DMA((NBUF,)),
            pltpu.SemaphoreType.DMA((NBUF,)),
        ],
    )(lgrg, x)
